# baseline (speedup 1.0000x reference)
import jax
import jax.numpy as jnp
from jax.experimental import pallas as pl

_B, _C, _Q, _F = 8, 192, 256, 1024
_K = 192
_M = _Q - _K + 1
_FB = 128
_NC = _F // _FB


def _attn_kernel(qw_ref, x_ref, out_ref):
    qw = qw_ref[...]
    qn = qw / jnp.maximum(jnp.sqrt(jnp.sum(qw * qw, axis=1, keepdims=True)), 1e-12)

    x4 = x_ref[0]
    m = jnp.int32(_M)
    neg_base = jnp.int32(jnp.iinfo(jnp.int32).min) + jnp.int32(1 << 30)

    s_tot = jnp.zeros((_Q, 1), jnp.float32)
    acc = jnp.zeros((_Q, _C), jnp.float32)
    for j in range(_NC):
        xc = jax.lax.slice(x4, (0, 4 * j, 0), (_C, 4 * j + 4, 32))
        xc = xc.reshape(_C, _FB)
        n = jnp.sqrt(jnp.sum(xc * xc, axis=0, keepdims=True))
        xn = xc / jnp.maximum(n, 1e-12)
        om = jnp.dot(qn, xn, preferred_element_type=jnp.float32)

        i = jax.lax.bitcast_convert_type(om, jnp.int32)
        key = i ^ (jax.lax.shift_right_arithmetic(i, 31) & jnp.int32(0x7FFFFFFF))

        def _count_ge(c):
            ind = jnp.where(key >= c, jnp.int32(1), jnp.int32(0))
            return jnp.sum(ind, axis=0, keepdims=True)

        cnt = _count_ge(jnp.zeros((1, _FB), jnp.int32))
        a = jnp.where(cnt >= m, jnp.int32(0), neg_base)
        a = jnp.broadcast_to(a, (1, _FB))
        for bit in range(29, -1, -1):
            c = a + jnp.int32(1 << bit)
            a = jnp.where(_count_ge(c) >= m, c, a)

        kth = jax.lax.bitcast_convert_type(
            a ^ (jax.lax.shift_right_arithmetic(a, 31) & jnp.int32(0x7FFFFFFF)),
            jnp.float32)

        keep = (om - kth) >= 0
        e = jnp.where(keep, jnp.exp(om - 1.0), 0.0)
        s_tot = s_tot + jnp.sum(e, axis=1, keepdims=True)
        acc = acc + jax.lax.dot_general(
            e, xn, (((1,), (1,)), ((), ())), preferred_element_type=jnp.float32)

    out_ref[0] = acc * (1.0 / s_tot)


@jax.jit
def kernel(input_features, query_weight):
    fn = pl.pallas_call(
        _attn_kernel,
        grid=(_B,),
        in_specs=[
            pl.BlockSpec((_Q, _C), lambda b: (0, 0)),
            pl.BlockSpec((1, _C, 32, 32), lambda b: (b, 0, 0, 0)),
        ],
        out_specs=pl.BlockSpec((1, _Q, _C), lambda b: (b, 0, 0)),
        out_shape=jax.ShapeDtypeStruct((_B, _Q, _C), jnp.float32),
    )
    return fn(query_weight, input_features)
